# baseline (speedup 1.0000x reference)
import jax
import jax.numpy as jnp
from jax import lax
from jax.experimental import pallas as pl
from jax.experimental.pallas import tpu as pltpu
from jax.experimental.pallas import tpu_sc as plsc

N = 10000
NP = 10240
E = 320000
D_IN = 128
HID = 32
HALF = 16
D_OUT = 64
ORDER = 8

NCORES = 2
NTILES = 16
ROWS = NP // NTILES
EPT = E // NTILES
CHUNK = 400
NCHUNKS = EPT // CHUNK


def _rsqrt_nr(d):
    y = jnp.full((16,), 0.0017, jnp.float32)
    for _ in range(28):
        y = y * (1.5 - 0.5 * d * y * y)
    return y


def _sc_body(h0_hbm, srcE_hbm, dstE_hbm, out_hbm, Gh,
             A, degS, degI,
             srcI, dstI, nA, nG, Yb, zcol, ones, nsb, ndb, eb0, eb1,
             gs0, gs1, ss0, ss1):
    c = lax.axis_index("c")
    s = lax.axis_index("s")
    rbase = s * ROWS
    rows_sl = pl.ds(rbase, ROWS)
    gbase = c * NP + rbase

    pltpu.sync_copy(srcE_hbm.at[s], srcI)
    pltpu.sync_copy(dstE_hbm.at[s], dstI)

    zv = jnp.zeros((16,), jnp.float32)
    ov = jnp.ones((16,), jnp.float32)
    for kk in range(32):
        ones[pl.ds(kk * 16, 16)] = ov
    for kk in range(ROWS // 16):
        zcol[pl.ds(kk * 16, 16)] = zv

    def zrow_body(i, carry):
        nG[i, :] = zv
        return carry
    lax.fori_loop(0, ROWS, zrow_body, 0)

    pltpu.sync_copy(zcol, degS.at[rows_sl])
    pltpu.sync_copy(zcol, degI.at[rows_sl])
    pltpu.sync_copy(nG, A.at[rows_sl, :])
    plsc.subcore_barrier()

    ones_c = ones.at[pl.ds(0, CHUNK)]

    def deg_body(j, carry):
        a = pltpu.async_copy(ones_c, degS.at[srcI.at[j]], gs0, add=True)
        b = pltpu.async_copy(ones_c, degI.at[dstI.at[j]], gs1, add=True)
        a.wait()
        b.wait()
        return carry
    lax.fori_loop(0, NCHUNKS, deg_body, 0)

    cnp = jnp.full((16,), c * NP, jnp.int32)

    def adj_body(j, carry):
        for kk in range(CHUNK // 16):
            sl = pl.ds(kk * 16, 16)
            srcI[j, sl] = srcI[j, sl] + cnp
        return carry
    lax.fori_loop(0, NCHUNKS, adj_body, 0)
    plsc.subcore_barrier()

    pltpu.sync_copy(degS.at[rows_sl], nsb)
    pltpu.sync_copy(degI.at[rows_sl], ndb)
    for kk in range(ROWS // 16):
        sl = pl.ds(kk * 16, 16)
        nsb[sl] = _rsqrt_nr(jnp.maximum(nsb[sl], 1.0))
        ndb[sl] = _rsqrt_nr(jnp.maximum(ndb[sl], 1.0))

    pltpu.sync_copy(h0_hbm.at[c, rows_sl, :], nA)

    def init_body(i, carry):
        iv = jnp.full((16,), i, jnp.int32)
        a = nA[i, :]
        Yb[i, :] = a
        nG[i, :] = a * plsc.load_gather(nsb, [iv])
        return carry
    lax.fori_loop(0, ROWS, init_body, 0)
    pltpu.sync_copy(nG, Gh.at[pl.ds(gbase, ROWS), :])
    plsc.subcore_barrier()

    for k in range(ORDER):
        pltpu.async_copy(Gh.at[srcI.at[0]], eb0, gs0)

        def edge_body(p, carry):
            j0 = 2 * p
            j1 = j0 + 1
            pltpu.async_copy(Gh.at[srcI.at[j1]], eb1, gs1)
            pltpu.make_async_copy(Gh.at[srcI.at[j0]], eb0, gs0).wait()
            s0 = pltpu.async_copy(eb0, A.at[dstI.at[j0]], ss0, add=True)
            pltpu.make_async_copy(Gh.at[srcI.at[j1]], eb1, gs1).wait()
            s1 = pltpu.async_copy(eb1, A.at[dstI.at[j1]], ss1, add=True)
            s0.wait()
            jn = jnp.minimum(j0 + 2, NCHUNKS - 1)
            pltpu.async_copy(Gh.at[srcI.at[jn]], eb0, gs0)
            s1.wait()
            return carry
        lax.fori_loop(0, NCHUNKS // 2, edge_body, 0)
        pltpu.make_async_copy(Gh.at[srcI.at[0]], eb0, gs0).wait()
        plsc.subcore_barrier()

        pltpu.sync_copy(A.at[rows_sl, :], nA)
        if k < ORDER - 1:
            def node_body(i, carry):
                iv = jnp.full((16,), i, jnp.int32)
                h = nA[i, :] * plsc.load_gather(ndb, [iv])
                Yb[i, :] = Yb[i, :] + h
                nG[i, :] = h * plsc.load_gather(nsb, [iv])
                return carry
            lax.fori_loop(0, ROWS, node_body, 0)
            pltpu.sync_copy(nG, Gh.at[pl.ds(gbase, ROWS), :])

            def rezero_body(i, carry):
                nG[i, :] = jnp.zeros((16,), jnp.float32)
                return carry
            lax.fori_loop(0, ROWS, rezero_body, 0)
            pltpu.sync_copy(nG, A.at[rows_sl, :])
            plsc.subcore_barrier()
        else:
            def node_last(i, carry):
                iv = jnp.full((16,), i, jnp.int32)
                Yb[i, :] = Yb[i, :] + nA[i, :] * plsc.load_gather(ndb, [iv])
                return carry
            lax.fori_loop(0, ROWS, node_last, 0)

    pltpu.sync_copy(Yb, out_hbm.at[c, rows_sl, :])


def _sc_propagate(h0_split, src3, dst3):
    mesh = plsc.VectorSubcoreMesh(core_axis_name="c", subcore_axis_name="s",
                                  num_cores=NCORES, num_subcores=NTILES)
    f32 = jnp.float32
    out, _ = pl.kernel(
        _sc_body,
        out_type=(jax.ShapeDtypeStruct((NCORES, NP, HALF), f32),
                  jax.ShapeDtypeStruct((NCORES * NP, HALF), f32)),
        mesh=mesh,
        compiler_params=pltpu.CompilerParams(needs_layout_passes=False,
                                             use_tc_tiling_on_sc=False),
        scratch_types=[
            pltpu.VMEM_SHARED((NP, HALF), f32),
            pltpu.VMEM_SHARED((NP,), f32),
            pltpu.VMEM_SHARED((NP,), f32),
            pltpu.VMEM((NCHUNKS, CHUNK), jnp.int32),
            pltpu.VMEM((NCHUNKS, CHUNK), jnp.int32),
            pltpu.VMEM((ROWS, HALF), f32),
            pltpu.VMEM((ROWS, HALF), f32),
            pltpu.VMEM((ROWS, HALF), f32),
            pltpu.VMEM((ROWS,), f32),
            pltpu.VMEM((512,), f32),
            pltpu.VMEM((ROWS,), f32),
            pltpu.VMEM((ROWS,), f32),
            pltpu.VMEM((CHUNK, HALF), f32),
            pltpu.VMEM((CHUNK, HALF), f32),
            pltpu.SemaphoreType.DMA,
            pltpu.SemaphoreType.DMA,
            pltpu.SemaphoreType.DMA,
            pltpu.SemaphoreType.DMA,
        ],
    )(h0_split, src3, dst3)
    return out


def _mm1_body(feats_ref, w1_ref, o_ref):
    o_ref[...] = jnp.dot(feats_ref[...], w1_ref[...],
                         preferred_element_type=jnp.float32) * (1.0 / 9.0)


def _mlp_body(y_ref, w2_ref, b1_ref, b2_ref, o_ref):
    a = jnp.maximum(y_ref[...] + b1_ref[...], 0.0)
    z = jnp.dot(a, w2_ref[...], preferred_element_type=jnp.float32) + b2_ref[...]
    m = jnp.max(z, axis=1, keepdims=True)
    lse = jnp.log(jnp.sum(jnp.exp(z - m), axis=1, keepdims=True)) + m
    o_ref[...] = z - lse


def kernel(feats, edge_index, W1, b1, W2, b2):
    src3 = edge_index[0].astype(jnp.int32).reshape(NTILES, NCHUNKS, CHUNK)
    dst3 = edge_index[1].astype(jnp.int32).reshape(NTILES, NCHUNKS, CHUNK)

    h0 = pl.pallas_call(
        _mm1_body,
        out_shape=jax.ShapeDtypeStruct((N, HID), jnp.float32),
    )(feats, W1)

    h0_split = jnp.pad(h0.reshape(N, NCORES, HALF).transpose(1, 0, 2),
                       ((0, 0), (0, NP - N), (0, 0)))

    y2 = _sc_propagate(h0_split, src3, dst3)
    y = jnp.concatenate([y2[0, :N, :], y2[1, :N, :]], axis=1)

    out = pl.pallas_call(
        _mlp_body,
        out_shape=jax.ShapeDtypeStruct((N, D_OUT), jnp.float32),
    )(y, W2, b1.reshape(1, HID), b2.reshape(1, D_OUT))

    return (out, out, out, out)


# speedup vs baseline: 1.1836x; 1.1836x over previous
import jax
import jax.numpy as jnp
from jax import lax
from jax.experimental import pallas as pl
from jax.experimental.pallas import tpu as pltpu
from jax.experimental.pallas import tpu_sc as plsc

N = 10000
NP = 10240
E = 320000
D_IN = 128
HID = 32
HALF = 16
D_OUT = 64
ORDER = 8

NCORES = 2
NTILES = 16
ROWS = NP // NTILES
EPT = E // NTILES
CHUNK = 250
NCHUNKS = EPT // CHUNK


def _rsqrt_nr(d):
    y = jnp.full((16,), 0.0029, jnp.float32)
    for _ in range(22):
        y = y * (1.5 - 0.5 * d * y * y)
    return y


def _sc_body(h0_hbm, srcE_hbm, dstE_hbm, out_hbm,
             A, Gh, degS, degI,
             srcI, dstI, nA, nG, Yb, zrows, zcol, ones, nsb, ndb,
             eb0, eb1, eb2, eb3,
             gs0, gs1, gs2, gs3, ss0, ss1, ss2, ss3):
    ebs = (eb0, eb1, eb2, eb3)
    gss = (gs0, gs1, gs2, gs3)
    sss = (ss0, ss1, ss2, ss3)
    c = lax.axis_index("c")
    s = lax.axis_index("s")
    rbase = s * ROWS
    rows_sl = pl.ds(rbase, ROWS)
    gbase = rbase

    pltpu.sync_copy(srcE_hbm.at[s], srcI)
    pltpu.sync_copy(dstE_hbm.at[s], dstI)

    zv = jnp.zeros((16,), jnp.float32)
    ov = jnp.ones((16,), jnp.float32)
    for kk in range(32):
        ones[pl.ds(kk * 16, 16)] = ov
    for kk in range(ROWS // 16):
        zcol[pl.ds(kk * 16, 16)] = zv

    def zrow_body(i, carry):
        zrows[i, :] = zv
        return carry
    lax.fori_loop(0, ROWS, zrow_body, 0, unroll=4)

    pltpu.sync_copy(zcol, degS.at[rows_sl])
    pltpu.sync_copy(zcol, degI.at[rows_sl])
    pltpu.sync_copy(zrows, A.at[rows_sl, :])
    plsc.subcore_barrier()

    ones_c = ones.at[pl.ds(0, CHUNK)]

    def deg_body(p, carry):
        j = 4 * p
        for b in range(4):
            pltpu.async_copy(ones_c, degS.at[srcI.at[j + b]], gs0, add=True)
            pltpu.async_copy(ones_c, degI.at[dstI.at[j + b]], gs1, add=True)
        for b in range(4):
            pltpu.make_async_copy(ones_c, degS.at[srcI.at[j]], gs0).wait()
            pltpu.make_async_copy(ones_c, degI.at[dstI.at[j]], gs1).wait()
        return carry
    lax.fori_loop(0, NCHUNKS // 4, deg_body, 0)

    plsc.subcore_barrier()

    pltpu.sync_copy(degS.at[rows_sl], nsb)
    pltpu.sync_copy(degI.at[rows_sl], ndb)
    for kk in range(ROWS // 16):
        sl = pl.ds(kk * 16, 16)
        nsb[sl] = _rsqrt_nr(jnp.maximum(nsb[sl], 1.0))
        ndb[sl] = _rsqrt_nr(jnp.maximum(ndb[sl], 1.0))

    pltpu.sync_copy(h0_hbm.at[c, rows_sl, :], nA)

    def init_body(i, carry):
        iv = jnp.full((16,), i, jnp.int32)
        a = nA[i, :]
        Yb[i, :] = a
        nG[i, :] = a * plsc.load_gather(nsb, [iv])
        return carry
    lax.fori_loop(0, ROWS, init_body, 0, unroll=4)
    pltpu.sync_copy(nG, Gh.at[pl.ds(gbase, ROWS), :])
    plsc.subcore_barrier()

    for k in range(ORDER):
        for b in range(4):
            pltpu.async_copy(Gh.at[srcI.at[b]], ebs[b], gss[b])

        def edge_body(p, carry):
            j = 4 * p
            scats = []
            for b in range(4):
                pltpu.make_async_copy(Gh.at[srcI.at[j + b]], ebs[b],
                                      gss[b]).wait()
                scats.append(pltpu.async_copy(ebs[b], A.at[dstI.at[j + b]],
                                              sss[b], add=True))
            for b in range(4):
                scats[b].wait()
                jn = jnp.minimum(j + 4 + b, NCHUNKS - 1)
                pltpu.async_copy(Gh.at[srcI.at[jn]], ebs[b], gss[b])
            return carry
        lax.fori_loop(0, NCHUNKS // 4, edge_body, 0)
        for b in range(4):
            pltpu.make_async_copy(Gh.at[srcI.at[0]], ebs[b], gss[b]).wait()
        plsc.subcore_barrier()

        pltpu.sync_copy(A.at[rows_sl, :], nA)
        if k < ORDER - 1:
            def node_body(i, carry):
                iv = jnp.full((16,), i, jnp.int32)
                h = nA[i, :] * plsc.load_gather(ndb, [iv])
                Yb[i, :] = Yb[i, :] + h
                nG[i, :] = h * plsc.load_gather(nsb, [iv])
                return carry
            lax.fori_loop(0, ROWS, node_body, 0, unroll=4)
            pltpu.sync_copy(nG, Gh.at[pl.ds(gbase, ROWS), :])
            pltpu.sync_copy(zrows, A.at[rows_sl, :])
            plsc.subcore_barrier()
        else:
            def node_last(i, carry):
                iv = jnp.full((16,), i, jnp.int32)
                Yb[i, :] = Yb[i, :] + nA[i, :] * plsc.load_gather(ndb, [iv])
                return carry
            lax.fori_loop(0, ROWS, node_last, 0, unroll=4)

    pltpu.sync_copy(Yb, out_hbm.at[c, rows_sl, :])


def _sc_propagate(h0_split, src3, dst3):
    mesh = plsc.VectorSubcoreMesh(core_axis_name="c", subcore_axis_name="s",
                                  num_cores=NCORES, num_subcores=NTILES)
    f32 = jnp.float32
    out = pl.kernel(
        _sc_body,
        out_type=jax.ShapeDtypeStruct((NCORES, NP, HALF), f32),
        mesh=mesh,
        compiler_params=pltpu.CompilerParams(needs_layout_passes=False,
                                             use_tc_tiling_on_sc=False),
        scratch_types=[
            pltpu.VMEM_SHARED((NP, HALF), f32),
            pltpu.VMEM_SHARED((NP, HALF), f32),
            pltpu.VMEM_SHARED((NP,), f32),
            pltpu.VMEM_SHARED((NP,), f32),
            pltpu.VMEM((NCHUNKS, CHUNK), jnp.int32),
            pltpu.VMEM((NCHUNKS, CHUNK), jnp.int32),
            pltpu.VMEM((ROWS, HALF), f32),
            pltpu.VMEM((ROWS, HALF), f32),
            pltpu.VMEM((ROWS, HALF), f32),
            pltpu.VMEM((ROWS, HALF), f32),
            pltpu.VMEM((ROWS,), f32),
            pltpu.VMEM((512,), f32),
            pltpu.VMEM((ROWS,), f32),
            pltpu.VMEM((ROWS,), f32),
            pltpu.VMEM((CHUNK, HALF), f32),
            pltpu.VMEM((CHUNK, HALF), f32),
            pltpu.VMEM((CHUNK, HALF), f32),
            pltpu.VMEM((CHUNK, HALF), f32),
            pltpu.SemaphoreType.DMA,
            pltpu.SemaphoreType.DMA,
            pltpu.SemaphoreType.DMA,
            pltpu.SemaphoreType.DMA,
            pltpu.SemaphoreType.DMA,
            pltpu.SemaphoreType.DMA,
            pltpu.SemaphoreType.DMA,
            pltpu.SemaphoreType.DMA,
        ],
    )(h0_split, src3, dst3)
    return out


def _mm1_body(feats_ref, w1_ref, o_ref):
    h = jnp.dot(feats_ref[...], w1_ref[...],
                preferred_element_type=jnp.float32) * (1.0 / 9.0)
    zpad = jnp.zeros((NP - N, HALF), jnp.float32)
    o_ref[0, pl.ds(0, N), :] = h[:, 0:HALF]
    o_ref[1, pl.ds(0, N), :] = h[:, HALF:HID]
    o_ref[0, pl.ds(N, NP - N), :] = zpad
    o_ref[1, pl.ds(N, NP - N), :] = zpad


def _mlp_body(y_ref, w2_ref, b1_ref, b2_ref, o_ref):
    y = jnp.concatenate([y_ref[0, pl.ds(0, N), :], y_ref[1, pl.ds(0, N), :]],
                        axis=1)
    a = jnp.maximum(y + b1_ref[...], 0.0)
    z = jnp.dot(a, w2_ref[...], preferred_element_type=jnp.float32) + b2_ref[...]
    m = jnp.max(z, axis=1, keepdims=True)
    lse = jnp.log(jnp.sum(jnp.exp(z - m), axis=1, keepdims=True)) + m
    o_ref[...] = z - lse


def kernel(feats, edge_index, W1, b1, W2, b2):
    src3 = edge_index[0].astype(jnp.int32).reshape(NTILES, NCHUNKS, CHUNK)
    dst3 = edge_index[1].astype(jnp.int32).reshape(NTILES, NCHUNKS, CHUNK)

    h0_split = pl.pallas_call(
        _mm1_body,
        out_shape=jax.ShapeDtypeStruct((NCORES, NP, HALF), jnp.float32),
    )(feats, W1)

    y2 = _sc_propagate(h0_split, src3, dst3)

    out = pl.pallas_call(
        _mlp_body,
        out_shape=jax.ShapeDtypeStruct((N, D_OUT), jnp.float32),
    )(y2, W2, b1.reshape(1, HID), b2.reshape(1, D_OUT))

    return (out, out, out, out)


# speedup vs baseline: 1.2029x; 1.0163x over previous
import jax
import jax.numpy as jnp
from jax import lax
from jax.experimental import pallas as pl
from jax.experimental.pallas import tpu as pltpu
from jax.experimental.pallas import tpu_sc as plsc

N = 10000
NP = 10240
E = 320000
D_IN = 128
HID = 32
HALF = 16
D_OUT = 64
ORDER = 8

NCORES = 2
NTILES = 16
ROWS = NP // NTILES
EPT = E // NTILES
CHUNK = 250
NCHUNKS = EPT // CHUNK


def _rsqrt_nr(d):
    y = jnp.full((16,), 0.0029, jnp.float32)
    for _ in range(22):
        y = y * (1.5 - 0.5 * d * y * y)
    return y


def _sc_body(h0_hbm, srcE_hbm, dstE_hbm, out_hbm,
             A, Gh, degS, degI,
             srcI, dstI, nA, nG, Yb, zrows, zcol, ones, nsb, ndb,
             eb0, eb1, eb2, eb3,
             gs0, gs1, gs2, gs3, ss0, ss1, ss2, ss3):
    ebs = (eb0, eb1, eb2, eb3)
    gss = (gs0, gs1, gs2, gs3)
    sss = (ss0, ss1, ss2, ss3)
    c = lax.axis_index("c")
    s = lax.axis_index("s")
    rbase = s * ROWS
    rows_sl = pl.ds(rbase, ROWS)
    gbase = rbase

    pltpu.sync_copy(srcE_hbm.at[s], srcI)
    pltpu.sync_copy(dstE_hbm.at[s], dstI)

    zv = jnp.zeros((16,), jnp.float32)
    ov = jnp.ones((16,), jnp.float32)
    for kk in range(32):
        ones[pl.ds(kk * 16, 16)] = ov
    for kk in range(ROWS // 16):
        zcol[pl.ds(kk * 16, 16)] = zv

    def zrow_body(i, carry):
        zrows[i, :] = zv
        return carry
    lax.fori_loop(0, ROWS, zrow_body, 0, unroll=4)

    pltpu.sync_copy(zcol, degS.at[rows_sl])
    pltpu.sync_copy(zcol, degI.at[rows_sl])
    pltpu.sync_copy(zrows, A.at[rows_sl, :])
    plsc.subcore_barrier()

    ones_c = ones.at[pl.ds(0, CHUNK)]

    def deg_fire(j, carry):
        pltpu.async_copy(ones_c, degS.at[srcI.at[j]], gs0, add=True)
        pltpu.async_copy(ones_c, degI.at[dstI.at[j]], gs1, add=True)
        return carry
    lax.fori_loop(0, NCHUNKS, deg_fire, 0)

    def deg_drain(j, carry):
        pltpu.make_async_copy(ones_c, degS.at[srcI.at[j]], gs0).wait()
        pltpu.make_async_copy(ones_c, degI.at[dstI.at[j]], gs1).wait()
        return carry
    lax.fori_loop(0, NCHUNKS, deg_drain, 0)

    plsc.subcore_barrier()

    pltpu.sync_copy(degS.at[rows_sl], nsb)
    pltpu.sync_copy(degI.at[rows_sl], ndb)
    for kk in range(ROWS // 16):
        sl = pl.ds(kk * 16, 16)
        nsb[sl] = _rsqrt_nr(jnp.maximum(nsb[sl], 1.0))
        ndb[sl] = _rsqrt_nr(jnp.maximum(ndb[sl], 1.0))

    pltpu.sync_copy(h0_hbm.at[c, rows_sl, :], nA)

    def init_body(i, carry):
        iv = jnp.full((16,), i, jnp.int32)
        a = nA[i, :]
        Yb[i, :] = zv
        nG[i, :] = a * plsc.load_gather(nsb, [iv])
        return carry
    lax.fori_loop(0, ROWS, init_body, 0, unroll=4)
    pltpu.sync_copy(nA, out_hbm.at[c, rows_sl, :])
    pltpu.sync_copy(nG, Gh.at[pl.ds(gbase, ROWS), :])
    for kk in range(ROWS // 16):
        sl = pl.ds(kk * 16, 16)
        nsb[sl] = nsb[sl] * ndb[sl]
    plsc.subcore_barrier()

    for k in range(ORDER):
        for b in range(4):
            pltpu.async_copy(Gh.at[srcI.at[b]], ebs[b], gss[b])

        def edge_body(p, carry):
            j = 4 * p
            scats = []
            for b in range(4):
                pltpu.make_async_copy(Gh.at[srcI.at[j + b]], ebs[b],
                                      gss[b]).wait()
                scats.append(pltpu.async_copy(ebs[b], A.at[dstI.at[j + b]],
                                              sss[b], add=True))
            for b in range(4):
                scats[b].wait()
                jn = jnp.minimum(j + 4 + b, NCHUNKS - 1)
                pltpu.async_copy(Gh.at[srcI.at[jn]], ebs[b], gss[b])
            return carry
        lax.fori_loop(0, NCHUNKS // 4, edge_body, 0)
        for b in range(4):
            pltpu.make_async_copy(Gh.at[srcI.at[0]], ebs[b], gss[b]).wait()
        plsc.subcore_barrier()

        pltpu.sync_copy(A.at[rows_sl, :], nA)
        if k < ORDER - 1:
            def node_body(i, carry):
                iv = jnp.full((16,), i, jnp.int32)
                a = nA[i, :]
                Yb[i, :] = Yb[i, :] + a
                nG[i, :] = a * plsc.load_gather(nsb, [iv])
                return carry
            lax.fori_loop(0, ROWS, node_body, 0, unroll=4)
            cg = pltpu.async_copy(nG, Gh.at[pl.ds(gbase, ROWS), :], ss0)
            cz = pltpu.async_copy(zrows, A.at[rows_sl, :], ss1)
            cg.wait()
            cz.wait()
            plsc.subcore_barrier()
        else:
            pltpu.sync_copy(out_hbm.at[c, rows_sl, :], nG)

            def node_last(i, carry):
                iv = jnp.full((16,), i, jnp.int32)
                Yb[i, :] = nG[i, :] + plsc.load_gather(ndb, [iv]) * (
                    Yb[i, :] + nA[i, :])
                return carry
            lax.fori_loop(0, ROWS, node_last, 0, unroll=4)

    pltpu.sync_copy(Yb, out_hbm.at[c, rows_sl, :])


def _sc_propagate(h0_split, src3, dst3):
    mesh = plsc.VectorSubcoreMesh(core_axis_name="c", subcore_axis_name="s",
                                  num_cores=NCORES, num_subcores=NTILES)
    f32 = jnp.float32
    out = pl.kernel(
        _sc_body,
        out_type=jax.ShapeDtypeStruct((NCORES, NP, HALF), f32),
        mesh=mesh,
        compiler_params=pltpu.CompilerParams(needs_layout_passes=False,
                                             use_tc_tiling_on_sc=False),
        scratch_types=[
            pltpu.VMEM_SHARED((NP, HALF), f32),
            pltpu.VMEM_SHARED((NP, HALF), f32),
            pltpu.VMEM_SHARED((NP,), f32),
            pltpu.VMEM_SHARED((NP,), f32),
            pltpu.VMEM((NCHUNKS, CHUNK), jnp.int32),
            pltpu.VMEM((NCHUNKS, CHUNK), jnp.int32),
            pltpu.VMEM((ROWS, HALF), f32),
            pltpu.VMEM((ROWS, HALF), f32),
            pltpu.VMEM((ROWS, HALF), f32),
            pltpu.VMEM((ROWS, HALF), f32),
            pltpu.VMEM((ROWS,), f32),
            pltpu.VMEM((512,), f32),
            pltpu.VMEM((ROWS,), f32),
            pltpu.VMEM((ROWS,), f32),
            pltpu.VMEM((CHUNK, HALF), f32),
            pltpu.VMEM((CHUNK, HALF), f32),
            pltpu.VMEM((CHUNK, HALF), f32),
            pltpu.VMEM((CHUNK, HALF), f32),
            pltpu.SemaphoreType.DMA,
            pltpu.SemaphoreType.DMA,
            pltpu.SemaphoreType.DMA,
            pltpu.SemaphoreType.DMA,
            pltpu.SemaphoreType.DMA,
            pltpu.SemaphoreType.DMA,
            pltpu.SemaphoreType.DMA,
            pltpu.SemaphoreType.DMA,
        ],
    )(h0_split, src3, dst3)
    return out


def _mm1_body(feats_ref, w1_ref, o_ref):
    h = jnp.dot(feats_ref[...], w1_ref[...],
                preferred_element_type=jnp.float32) * (1.0 / 9.0)
    zpad = jnp.zeros((NP - N, HALF), jnp.float32)
    o_ref[0, pl.ds(0, N), :] = h[:, 0:HALF]
    o_ref[1, pl.ds(0, N), :] = h[:, HALF:HID]
    o_ref[0, pl.ds(N, NP - N), :] = zpad
    o_ref[1, pl.ds(N, NP - N), :] = zpad


def _mlp_body(y_ref, w2_ref, b1_ref, b2_ref, o_ref):
    y = jnp.concatenate([y_ref[0, pl.ds(0, N), :], y_ref[1, pl.ds(0, N), :]],
                        axis=1)
    a = jnp.maximum(y + b1_ref[...], 0.0)
    z = jnp.dot(a, w2_ref[...], preferred_element_type=jnp.float32) + b2_ref[...]
    m = jnp.max(z, axis=1, keepdims=True)
    lse = jnp.log(jnp.sum(jnp.exp(z - m), axis=1, keepdims=True)) + m
    o_ref[...] = z - lse


def kernel(feats, edge_index, W1, b1, W2, b2):
    src3 = edge_index[0].astype(jnp.int32).reshape(NTILES, NCHUNKS, CHUNK)
    dst3 = edge_index[1].astype(jnp.int32).reshape(NTILES, NCHUNKS, CHUNK)

    h0_split = pl.pallas_call(
        _mm1_body,
        out_shape=jax.ShapeDtypeStruct((NCORES, NP, HALF), jnp.float32),
    )(feats, W1)

    y2 = _sc_propagate(h0_split, src3, dst3)

    out = pl.pallas_call(
        _mlp_body,
        out_shape=jax.ShapeDtypeStruct((N, D_OUT), jnp.float32),
    )(y2, W2, b1.reshape(1, HID), b2.reshape(1, D_OUT))

    return (out, out, out, out)


# speedup vs baseline: 1.2061x; 1.0026x over previous
import jax
import jax.numpy as jnp
from jax import lax
from jax.experimental import pallas as pl
from jax.experimental.pallas import tpu as pltpu
from jax.experimental.pallas import tpu_sc as plsc

N = 10000
NP = 10240
E = 320000
D_IN = 128
HID = 32
HALF = 16
D_OUT = 64
ORDER = 8

NCORES = 2
NTILES = 16
ROWS = NP // NTILES
EPT = E // NTILES
CHUNK = 250
NCHUNKS = EPT // CHUNK


def _rsqrt_nr(d):
    y = jnp.full((16,), 0.0029, jnp.float32)
    for _ in range(22):
        y = y * (1.5 - 0.5 * d * y * y)
    return y


def _sc_body(h0_hbm, srcE_hbm, dstE_hbm, out_hbm,
             A, Gh, degS, degI,
             srcI, dstI, nA, nG, Yb, zrows, zcol, ones, nsb, ndb,
             eb0, eb1, eb2, eb3,
             gs0, gs1, gs2, gs3, ss0, ss1, ss2, ss3):
    ebs = (eb0, eb1, eb2, eb3)
    gss = (gs0, gs1, gs2, gs3)
    sss = (ss0, ss1, ss2, ss3)
    c = lax.axis_index("c")
    s = lax.axis_index("s")
    rbase = s * ROWS
    rows_sl = pl.ds(rbase, ROWS)
    gbase = rbase

    pltpu.sync_copy(srcE_hbm.at[s], srcI)
    pltpu.sync_copy(dstE_hbm.at[s], dstI)

    zv = jnp.zeros((16,), jnp.float32)
    ov = jnp.ones((16,), jnp.float32)
    for kk in range(32):
        ones[pl.ds(kk * 16, 16)] = ov
    for kk in range(ROWS // 16):
        zcol[pl.ds(kk * 16, 16)] = zv

    def zrow_body(i, carry):
        zrows[i, :] = zv
        return carry
    lax.fori_loop(0, ROWS, zrow_body, 0, unroll=4)

    pltpu.sync_copy(zcol, degS.at[rows_sl])
    pltpu.sync_copy(zcol, degI.at[rows_sl])
    pltpu.sync_copy(zrows, A.at[rows_sl, :])
    plsc.subcore_barrier()

    ones_c = ones.at[pl.ds(0, CHUNK)]

    def deg_fire(j, carry):
        pltpu.async_copy(ones_c, degS.at[srcI.at[j]], gs0, add=True)
        pltpu.async_copy(ones_c, degI.at[dstI.at[j]], gs1, add=True)
        return carry
    lax.fori_loop(0, NCHUNKS, deg_fire, 0)

    def deg_drain(j, carry):
        pltpu.make_async_copy(ones_c, degS.at[srcI.at[j]], gs0).wait()
        pltpu.make_async_copy(ones_c, degI.at[dstI.at[j]], gs1).wait()
        return carry
    lax.fori_loop(0, NCHUNKS, deg_drain, 0)

    plsc.subcore_barrier()

    pltpu.sync_copy(degS.at[rows_sl], nsb)
    pltpu.sync_copy(degI.at[rows_sl], ndb)
    for kk in range(ROWS // 16):
        sl = pl.ds(kk * 16, 16)
        nsb[sl] = _rsqrt_nr(jnp.maximum(nsb[sl], 1.0))
        ndb[sl] = _rsqrt_nr(jnp.maximum(ndb[sl], 1.0))

    pltpu.sync_copy(h0_hbm.at[c, rows_sl, :], nA)

    def init_body(i, carry):
        iv = jnp.full((16,), i, jnp.int32)
        a = nA[i, :]
        Yb[i, :] = zv
        nG[i, :] = a * plsc.load_gather(nsb, [iv])
        return carry
    lax.fori_loop(0, ROWS, init_body, 0, unroll=4)
    pltpu.sync_copy(nA, out_hbm.at[c, rows_sl, :])
    pltpu.sync_copy(nG, Gh.at[pl.ds(gbase, ROWS), :])
    for kk in range(ROWS // 16):
        sl = pl.ds(kk * 16, 16)
        nsb[sl] = nsb[sl] * ndb[sl]
    plsc.subcore_barrier()

    def hop_body(k, carry):
        for b in range(4):
            pltpu.async_copy(Gh.at[srcI.at[b]], ebs[b], gss[b])

        def edge_body(p, carry2):
            j = 4 * p
            scats = []
            for b in range(4):
                pltpu.make_async_copy(Gh.at[srcI.at[j + b]], ebs[b],
                                      gss[b]).wait()
                scats.append(pltpu.async_copy(ebs[b], A.at[dstI.at[j + b]],
                                              sss[b], add=True))
            for b in range(4):
                scats[b].wait()
                jn = jnp.minimum(j + 4 + b, NCHUNKS - 1)
                pltpu.async_copy(Gh.at[srcI.at[jn]], ebs[b], gss[b])
            return carry2
        lax.fori_loop(0, NCHUNKS // 4, edge_body, 0)
        for b in range(4):
            pltpu.make_async_copy(Gh.at[srcI.at[0]], ebs[b], gss[b]).wait()
        plsc.subcore_barrier()

        pltpu.sync_copy(A.at[rows_sl, :], nA)

        def node_body(i, carry2):
            iv = jnp.full((16,), i, jnp.int32)
            a = nA[i, :]
            Yb[i, :] = Yb[i, :] + a
            nG[i, :] = a * plsc.load_gather(nsb, [iv])
            return carry2
        lax.fori_loop(0, ROWS, node_body, 0, unroll=4)
        cg = pltpu.async_copy(nG, Gh.at[pl.ds(gbase, ROWS), :], ss0)
        cz = pltpu.async_copy(zrows, A.at[rows_sl, :], ss1)
        cg.wait()
        cz.wait()
        plsc.subcore_barrier()
        return carry
    lax.fori_loop(0, ORDER - 1, hop_body, 0)

    for b in range(4):
        pltpu.async_copy(Gh.at[srcI.at[b]], ebs[b], gss[b])

    def edge_body_f(p, carry):
        j = 4 * p
        scats = []
        for b in range(4):
            pltpu.make_async_copy(Gh.at[srcI.at[j + b]], ebs[b],
                                  gss[b]).wait()
            scats.append(pltpu.async_copy(ebs[b], A.at[dstI.at[j + b]],
                                          sss[b], add=True))
        for b in range(4):
            scats[b].wait()
            jn = jnp.minimum(j + 4 + b, NCHUNKS - 1)
            pltpu.async_copy(Gh.at[srcI.at[jn]], ebs[b], gss[b])
        return carry
    lax.fori_loop(0, NCHUNKS // 4, edge_body_f, 0)
    for b in range(4):
        pltpu.make_async_copy(Gh.at[srcI.at[0]], ebs[b], gss[b]).wait()
    plsc.subcore_barrier()

    pltpu.sync_copy(A.at[rows_sl, :], nA)
    pltpu.sync_copy(out_hbm.at[c, rows_sl, :], nG)

    def node_last(i, carry):
        iv = jnp.full((16,), i, jnp.int32)
        Yb[i, :] = nG[i, :] + plsc.load_gather(ndb, [iv]) * (
            Yb[i, :] + nA[i, :])
        return carry
    lax.fori_loop(0, ROWS, node_last, 0, unroll=4)

    pltpu.sync_copy(Yb, out_hbm.at[c, rows_sl, :])


def _sc_propagate(h0_split, src3, dst3):
    mesh = plsc.VectorSubcoreMesh(core_axis_name="c", subcore_axis_name="s",
                                  num_cores=NCORES, num_subcores=NTILES)
    f32 = jnp.float32
    out = pl.kernel(
        _sc_body,
        out_type=jax.ShapeDtypeStruct((NCORES, NP, HALF), f32),
        mesh=mesh,
        compiler_params=pltpu.CompilerParams(needs_layout_passes=False,
                                             use_tc_tiling_on_sc=False),
        scratch_types=[
            pltpu.VMEM_SHARED((NP, HALF), f32),
            pltpu.VMEM_SHARED((NP, HALF), f32),
            pltpu.VMEM_SHARED((NP,), f32),
            pltpu.VMEM_SHARED((NP,), f32),
            pltpu.VMEM((NCHUNKS, CHUNK), jnp.int32),
            pltpu.VMEM((NCHUNKS, CHUNK), jnp.int32),
            pltpu.VMEM((ROWS, HALF), f32),
            pltpu.VMEM((ROWS, HALF), f32),
            pltpu.VMEM((ROWS, HALF), f32),
            pltpu.VMEM((ROWS, HALF), f32),
            pltpu.VMEM((ROWS,), f32),
            pltpu.VMEM((512,), f32),
            pltpu.VMEM((ROWS,), f32),
            pltpu.VMEM((ROWS,), f32),
            pltpu.VMEM((CHUNK, HALF), f32),
            pltpu.VMEM((CHUNK, HALF), f32),
            pltpu.VMEM((CHUNK, HALF), f32),
            pltpu.VMEM((CHUNK, HALF), f32),
            pltpu.SemaphoreType.DMA,
            pltpu.SemaphoreType.DMA,
            pltpu.SemaphoreType.DMA,
            pltpu.SemaphoreType.DMA,
            pltpu.SemaphoreType.DMA,
            pltpu.SemaphoreType.DMA,
            pltpu.SemaphoreType.DMA,
            pltpu.SemaphoreType.DMA,
        ],
    )(h0_split, src3, dst3)
    return out


def _mm1_body(feats_ref, w1_ref, o_ref):
    h = jnp.dot(feats_ref[...], w1_ref[...],
                preferred_element_type=jnp.float32) * (1.0 / 9.0)
    zpad = jnp.zeros((NP - N, HALF), jnp.float32)
    o_ref[0, pl.ds(0, N), :] = h[:, 0:HALF]
    o_ref[1, pl.ds(0, N), :] = h[:, HALF:HID]
    o_ref[0, pl.ds(N, NP - N), :] = zpad
    o_ref[1, pl.ds(N, NP - N), :] = zpad


def _mlp_body(y_ref, w2_ref, b1_ref, b2_ref, o_ref):
    y = jnp.concatenate([y_ref[0, pl.ds(0, N), :], y_ref[1, pl.ds(0, N), :]],
                        axis=1)
    a = jnp.maximum(y + b1_ref[...], 0.0)
    z = jnp.dot(a, w2_ref[...], preferred_element_type=jnp.float32) + b2_ref[...]
    m = jnp.max(z, axis=1, keepdims=True)
    lse = jnp.log(jnp.sum(jnp.exp(z - m), axis=1, keepdims=True)) + m
    o_ref[...] = z - lse


def kernel(feats, edge_index, W1, b1, W2, b2):
    src3 = edge_index[0].astype(jnp.int32).reshape(NTILES, NCHUNKS, CHUNK)
    dst3 = edge_index[1].astype(jnp.int32).reshape(NTILES, NCHUNKS, CHUNK)

    h0_split = pl.pallas_call(
        _mm1_body,
        out_shape=jax.ShapeDtypeStruct((NCORES, NP, HALF), jnp.float32),
    )(feats, W1)

    y2 = _sc_propagate(h0_split, src3, dst3)

    out = pl.pallas_call(
        _mlp_body,
        out_shape=jax.ShapeDtypeStruct((N, D_OUT), jnp.float32),
    )(y2, W2, b1.reshape(1, HID), b2.reshape(1, D_OUT))

    return (out, out, out, out)
